# Initial kernel scaffold; baseline (speedup 1.0000x reference)
import jax
import jax.numpy as jnp
from jax import lax
from jax.experimental import pallas as pl
from jax.experimental.pallas import tpu as pltpu
from jax.experimental.pallas import tpu_sc as plsc

NUM_CLASSES = 21
NBINS = NUM_CLASSES * NUM_CLASSES
HBINS = 448
LANES = 16
NWORKERS = 32
N_PIXELS = 16 * 512 * 512
PER_WORKER = N_PIXELS // NWORKERS
CHUNK = 16384
NCHUNKS = PER_WORKER // CHUNK
VECS_PER_CHUNK = CHUNK // LANES


def _body(pred_hbm, lab_hbm, out_hbm, pred_v, lab_v, hist_v, merged_v, sem0, sem1):
    wid = lax.axis_index("s") * 2 + lax.axis_index("c")
    base = wid * PER_WORKER

    lane = lax.iota(jnp.int32, LANES)
    lane_base = lane * HBINS
    ones = jnp.full((LANES,), 1, jnp.int32)
    zeros = jnp.zeros((LANES,), jnp.int32)

    def zero_step(j, carry):
        hist_v[pl.ds(j * LANES, LANES)] = zeros
        return carry

    lax.fori_loop(0, (LANES * HBINS) // LANES, zero_step, 0)

    pltpu.async_copy(pred_hbm.at[pl.ds(base, CHUNK)], pred_v.at[0], sem0)
    pltpu.async_copy(lab_hbm.at[pl.ds(base, CHUNK)], lab_v.at[0], sem1)

    def chunk_step(c, carry):
        buf = lax.rem(c, 2)
        nxt = lax.rem(c + 1, 2)

        @pl.when(c + 1 < NCHUNKS)
        def _():
            pltpu.async_copy(
                pred_hbm.at[pl.ds(base + (c + 1) * CHUNK, CHUNK)],
                pred_v.at[nxt], sem0)
            pltpu.async_copy(
                lab_hbm.at[pl.ds(base + (c + 1) * CHUNK, CHUNK)],
                lab_v.at[nxt], sem1)

        pltpu.make_async_copy(pred_hbm.at[pl.ds(base, CHUNK)],
                              pred_v.at[buf], sem0).wait()
        pltpu.make_async_copy(lab_hbm.at[pl.ds(base, CHUNK)],
                              lab_v.at[buf], sem1).wait()

        def vec_step(i, carry2):
            off = i * LANES
            p = pred_v[buf, pl.ds(off, LANES)]
            l = lab_v[buf, pl.ds(off, LANES)]
            idx = lane_base + l * NUM_CLASSES + p
            plsc.addupdate_scatter(hist_v, [idx], ones)
            return carry2

        lax.fori_loop(0, VECS_PER_CHUNK, vec_step, 0)
        return carry

    lax.fori_loop(0, NCHUNKS, chunk_step, 0)

    def merge_step(j, carry):
        off = j * LANES
        acc = hist_v[pl.ds(off, LANES)]
        for ln in range(1, LANES):
            acc = acc + hist_v[pl.ds(ln * HBINS + off, LANES)]
        merged_v[pl.ds(off, LANES)] = acc
        return carry

    lax.fori_loop(0, HBINS // LANES, merge_step, 0)

    pltpu.sync_copy(merged_v, out_hbm.at[wid])


@jax.jit
def _run(pred_flat, lab_flat):
    mesh = plsc.VectorSubcoreMesh(core_axis_name="c", subcore_axis_name="s")
    partials = pl.kernel(
        _body,
        mesh=mesh,
        out_type=jax.ShapeDtypeStruct((NWORKERS, HBINS), jnp.int32),
        scratch_types=[
            pltpu.VMEM((2, CHUNK), jnp.int32),
            pltpu.VMEM((2, CHUNK), jnp.int32),
            pltpu.VMEM((LANES * HBINS,), jnp.int32),
            pltpu.VMEM((HBINS,), jnp.int32),
            pltpu.SemaphoreType.DMA,
            pltpu.SemaphoreType.DMA,
        ],
    )(pred_flat, lab_flat)
    return partials.sum(axis=0)[:NBINS].reshape(NUM_CLASSES, NUM_CLASSES)


def kernel(imgPredict, imgLabel):
    pred_flat = imgPredict.reshape(-1)
    lab_flat = imgLabel.reshape(-1)
    return _run(pred_flat, lab_flat)


# speedup vs baseline: 1.2581x; 1.2581x over previous
import jax
import jax.numpy as jnp
from jax import lax
from jax.experimental import pallas as pl
from jax.experimental.pallas import tpu as pltpu
from jax.experimental.pallas import tpu_sc as plsc

NUM_CLASSES = 21
NBINS = NUM_CLASSES * NUM_CLASSES
HBINS = 448
LANES = 16
NWORKERS = 32
N_PIXELS = 16 * 512 * 512
PER_WORKER = N_PIXELS // NWORKERS
CHUNK = 16384
NCHUNKS = PER_WORKER // CHUNK
VECS_PER_CHUNK = CHUNK // LANES


def _body(pred_hbm, lab_hbm, out_hbm, pred_v, lab_v, hist_v, merged_v, sem0, sem1):
    wid = lax.axis_index("s") * 2 + lax.axis_index("c")
    base = wid * PER_WORKER

    lane = lax.iota(jnp.int32, LANES)
    lane_base = lane * HBINS
    ones = jnp.full((LANES,), 1, jnp.int32)
    zeros = jnp.zeros((LANES,), jnp.int32)

    def zero_step(j, carry):
        hist_v[pl.ds(j * LANES, LANES)] = zeros
        return carry

    lax.fori_loop(0, (LANES * HBINS) // LANES, zero_step, 0)

    pltpu.async_copy(pred_hbm.at[pl.ds(base, CHUNK)], pred_v.at[0], sem0)
    pltpu.async_copy(lab_hbm.at[pl.ds(base, CHUNK)], lab_v.at[0], sem1)

    def chunk_step(c, carry):
        buf = lax.rem(c, 2)
        nxt = lax.rem(c + 1, 2)

        @pl.when(c + 1 < NCHUNKS)
        def _():
            pltpu.async_copy(
                pred_hbm.at[pl.ds(base + (c + 1) * CHUNK, CHUNK)],
                pred_v.at[nxt], sem0)
            pltpu.async_copy(
                lab_hbm.at[pl.ds(base + (c + 1) * CHUNK, CHUNK)],
                lab_v.at[nxt], sem1)

        pltpu.make_async_copy(pred_hbm.at[pl.ds(base, CHUNK)],
                              pred_v.at[buf], sem0).wait()
        pltpu.make_async_copy(lab_hbm.at[pl.ds(base, CHUNK)],
                              lab_v.at[buf], sem1).wait()

        def vec_step(i, carry2):
            off = i * LANES
            p = pred_v[buf, pl.ds(off, LANES)]
            l = lab_v[buf, pl.ds(off, LANES)]
            idx = lane_base + l * NUM_CLASSES + p
            plsc.addupdate_scatter(hist_v, [idx], ones)
            return carry2

        lax.fori_loop(0, VECS_PER_CHUNK, vec_step, 0)
        return carry

    lax.fori_loop(0, NCHUNKS, chunk_step, 0)

    def merge_step(j, carry):
        off = j * LANES
        acc = hist_v[pl.ds(off, LANES)]
        for ln in range(1, LANES):
            acc = acc + hist_v[pl.ds(ln * HBINS + off, LANES)]
        merged_v[pl.ds(off, LANES)] = acc
        return carry

    lax.fori_loop(0, HBINS // LANES, merge_step, 0)

    pltpu.sync_copy(merged_v, out_hbm.at[wid])


@jax.jit
def _run(pred_flat, lab_flat):
    mesh = plsc.VectorSubcoreMesh(core_axis_name="c", subcore_axis_name="s")
    partials = pl.kernel(
        _body,
        mesh=mesh,
        out_type=jax.ShapeDtypeStruct((NWORKERS, HBINS), jnp.int32),
        compiler_params=pltpu.CompilerParams(needs_layout_passes=False),
        scratch_types=[
            pltpu.VMEM((2, CHUNK), jnp.int32),
            pltpu.VMEM((2, CHUNK), jnp.int32),
            pltpu.VMEM((LANES * HBINS,), jnp.int32),
            pltpu.VMEM((HBINS,), jnp.int32),
            pltpu.SemaphoreType.DMA,
            pltpu.SemaphoreType.DMA,
        ],
    )(pred_flat, lab_flat)
    return partials.sum(axis=0)[:NBINS].reshape(NUM_CLASSES, NUM_CLASSES)


def kernel(imgPredict, imgLabel):
    pred_flat = imgPredict.reshape(-1)
    lab_flat = imgLabel.reshape(-1)
    return _run(pred_flat, lab_flat)


# speedup vs baseline: 2.3854x; 1.8960x over previous
import jax
import jax.numpy as jnp
from jax import lax
from jax.experimental import pallas as pl
from jax.experimental.pallas import tpu as pltpu
from jax.experimental.pallas import tpu_sc as plsc

NUM_CLASSES = 21
NBINS = NUM_CLASSES * NUM_CLASSES
HBINS = 448
LANES = 16
NWORKERS = 32
N_PIXELS = 16 * 512 * 512
PER_WORKER = N_PIXELS // NWORKERS
CHUNK = 16384
NCHUNKS = PER_WORKER // CHUNK
VECS_PER_CHUNK = CHUNK // LANES
UNROLL = 8


def _body(pred_hbm, lab_hbm, out_hbm,
          pred_a, pred_b, lab_a, lab_b, hist_v, merged_v, sem0, sem1):
    wid = lax.axis_index("s") * 2 + lax.axis_index("c")
    base = wid * PER_WORKER

    lane = lax.iota(jnp.int32, LANES)
    lane_base = lane * HBINS
    ones = jnp.full((LANES,), 1, jnp.int32)
    zeros = jnp.zeros((LANES,), jnp.int32)

    @plsc.parallel_loop(0, (LANES * HBINS) // LANES)
    def _(j):
        hist_v[pl.ds(j * LANES, LANES)] = zeros

    pred_bufs = (pred_a, pred_b)
    lab_bufs = (lab_a, lab_b)

    pltpu.async_copy(pred_hbm.at[pl.ds(base, CHUNK)], pred_a, sem0)
    pltpu.async_copy(lab_hbm.at[pl.ds(base, CHUNK)], lab_a, sem1)

    for c in range(NCHUNKS):
        pv = pred_bufs[c % 2]
        lv = lab_bufs[c % 2]

        if c + 1 < NCHUNKS:
            pltpu.async_copy(
                pred_hbm.at[pl.ds(base + (c + 1) * CHUNK, CHUNK)],
                pred_bufs[(c + 1) % 2], sem0)
            pltpu.async_copy(
                lab_hbm.at[pl.ds(base + (c + 1) * CHUNK, CHUNK)],
                lab_bufs[(c + 1) % 2], sem1)

        pltpu.make_async_copy(pred_hbm.at[pl.ds(base, CHUNK)], pv, sem0).wait()
        pltpu.make_async_copy(lab_hbm.at[pl.ds(base, CHUNK)], lv, sem1).wait()

        @plsc.parallel_loop(0, VECS_PER_CHUNK, unroll=UNROLL)
        def _(i):
            off = i * LANES
            p = pv[pl.ds(off, LANES)]
            l = lv[pl.ds(off, LANES)]
            idx = lane_base + l * NUM_CLASSES + p
            plsc.addupdate_scatter(hist_v, [idx], ones)

    @plsc.parallel_loop(0, HBINS // LANES)
    def _(j):
        off = j * LANES
        acc = hist_v[pl.ds(off, LANES)]
        for ln in range(1, LANES):
            acc = acc + hist_v[pl.ds(ln * HBINS + off, LANES)]
        merged_v[pl.ds(off, LANES)] = acc

    pltpu.sync_copy(merged_v, out_hbm.at[wid])


@jax.jit
def _run(pred_flat, lab_flat):
    mesh = plsc.VectorSubcoreMesh(core_axis_name="c", subcore_axis_name="s")
    partials = pl.kernel(
        _body,
        mesh=mesh,
        out_type=jax.ShapeDtypeStruct((NWORKERS, HBINS), jnp.int32),
        compiler_params=pltpu.CompilerParams(needs_layout_passes=False),
        scratch_types=[
            pltpu.VMEM((CHUNK,), jnp.int32),
            pltpu.VMEM((CHUNK,), jnp.int32),
            pltpu.VMEM((CHUNK,), jnp.int32),
            pltpu.VMEM((CHUNK,), jnp.int32),
            pltpu.VMEM((LANES * HBINS,), jnp.int32),
            pltpu.VMEM((HBINS,), jnp.int32),
            pltpu.SemaphoreType.DMA,
            pltpu.SemaphoreType.DMA,
        ],
    )(pred_flat, lab_flat)
    return partials.sum(axis=0)[:NBINS].reshape(NUM_CLASSES, NUM_CLASSES)


def kernel(imgPredict, imgLabel):
    pred_flat = imgPredict.reshape(-1)
    lab_flat = imgLabel.reshape(-1)
    return _run(pred_flat, lab_flat)
